# baseline (speedup 1.0000x reference)
import jax
import jax.numpy as jnp
from jax.experimental import pallas as pl

NUM_CLASSES = 1000
ROWS = 16384 * 26
BLK = 1024


def _onehot_body(idx_ref, out_ref):
    idx = idx_ref[...]
    classes = jax.lax.broadcasted_iota(jnp.int32, (BLK, NUM_CLASSES), 1)
    out_ref[...] = idx == classes


def kernel(index):
    flat = index.reshape(ROWS, 1)
    out = pl.pallas_call(
        _onehot_body,
        grid=(ROWS // BLK,),
        in_specs=[pl.BlockSpec((BLK, 1), lambda i: (i, 0))],
        out_specs=pl.BlockSpec((BLK, NUM_CLASSES), lambda i: (i, 0)),
        out_shape=jax.ShapeDtypeStruct((ROWS, NUM_CLASSES), jnp.bool_),
    )(flat)
    return out.reshape(16384, 26, NUM_CLASSES)


# speedup vs baseline: 1.2161x; 1.2161x over previous
import jax
import jax.numpy as jnp
from jax.experimental import pallas as pl

NUM_CLASSES = 1000
N = 16384
M = 26
BLK = 128


def _onehot_body(idx_ref, out_ref):
    idx = idx_ref[...]
    classes = jax.lax.broadcasted_iota(jnp.int32, (BLK, M, NUM_CLASSES), 2)
    out_ref[...] = idx[..., None] == classes


def kernel(index):
    return pl.pallas_call(
        _onehot_body,
        grid=(N // BLK,),
        in_specs=[pl.BlockSpec((BLK, M), lambda i: (i, 0))],
        out_specs=pl.BlockSpec((BLK, M, NUM_CLASSES), lambda i: (i, 0, 0)),
        out_shape=jax.ShapeDtypeStruct((N, M, NUM_CLASSES), jnp.bool_),
    )(index)


# speedup vs baseline: 11.9693x; 9.8421x over previous
import jax
import jax.numpy as jnp
from jax.experimental import pallas as pl

NUM_CLASSES = 1000
N = 16384
M = 26
IB = 4096


def _onehot_body(idx_ref, out_ref):
    idx = idx_ref[...][0]
    word_idx = idx >> 2
    val = jnp.left_shift(jnp.int32(1), 8 * (idx & 3))
    w_iota = jax.lax.broadcasted_iota(jnp.int32, (1, NUM_CLASSES // 4, IB), 1)
    words = jnp.where(word_idx[:, None, :] == w_iota, val[:, None, :], 0)
    out_ref.bitcast(jnp.int32)[...] = words


def kernel(index):
    idx_t = index.T.reshape(M, 1, N)
    oh_t = pl.pallas_call(
        _onehot_body,
        grid=(M, N // IB),
        in_specs=[pl.BlockSpec((1, 1, IB), lambda j, i: (j, 0, i))],
        out_specs=pl.BlockSpec((1, NUM_CLASSES, IB), lambda j, i: (j, 0, i)),
        out_shape=jax.ShapeDtypeStruct((M, NUM_CLASSES, N), jnp.int8),
    )(idx_t)
    return oh_t.transpose(2, 0, 1).astype(jnp.bool_)


# speedup vs baseline: 12.2768x; 1.0257x over previous
import jax
import jax.numpy as jnp
from jax.experimental import pallas as pl

NUM_CLASSES = 1000
N = 16384
M = 26
IB = 8192


def _onehot_body(idx_ref, out_ref):
    idx = idx_ref[...][0]
    word_idx = idx >> 2
    val = jnp.left_shift(jnp.int32(1), 8 * (idx & 3))
    w_iota = jax.lax.broadcasted_iota(jnp.int32, (1, NUM_CLASSES // 4, IB), 1)
    words = jnp.where(word_idx[:, None, :] == w_iota, val[:, None, :], 0)
    out_ref.bitcast(jnp.int32)[...] = words


def kernel(index):
    idx_t = index.T.reshape(M, 1, N)
    oh_t = pl.pallas_call(
        _onehot_body,
        grid=(M, N // IB),
        in_specs=[pl.BlockSpec((1, 1, IB), lambda j, i: (j, 0, i))],
        out_specs=pl.BlockSpec((1, NUM_CLASSES, IB), lambda j, i: (j, 0, i)),
        out_shape=jax.ShapeDtypeStruct((M, NUM_CLASSES, N), jnp.int8),
    )(idx_t)
    return oh_t.transpose(2, 0, 1).astype(jnp.bool_)
